# baseline (speedup 1.0000x reference)
import jax
import jax.numpy as jnp
from jax.experimental import pallas as pl

NUM_CLASSES = 20
OH_ROWS = 32

BLK = 2048


def _proto_body(nsteps):
    def body(f_ref, l_ref, o_ref):
        b = pl.program_id(0)
        j = pl.program_id(1)
        step = b * pl.num_programs(1) + j

        f = f_ref[0]
        lab = l_ref[0]

        sumsq = jnp.sum(f * f, axis=0, keepdims=True)
        invn = 1.0 / jnp.maximum(jnp.sqrt(sumsq), 1e-12)

        kiota = jax.lax.broadcasted_iota(jnp.int32, (OH_ROWS, BLK), 0)
        oh = jnp.where(kiota == lab, invn, 0.0)

        partial = jax.lax.dot_general(
            f, oh, (((1,), (1,)), ((), ())),
            preferred_element_type=jnp.float32)

        @pl.when(step == 0)
        def _():
            o_ref[...] = jnp.zeros_like(o_ref)

        o_ref[...] += partial

        @pl.when(step == nsteps - 1)
        def _():
            p = o_ref[...]
            pn = jnp.sqrt(jnp.sum(p * p, axis=0, keepdims=True))
            o_ref[...] = p / jnp.maximum(pn, 1e-12)

    return body


def kernel(features, labels):
    bs, c, h, w = features.shape
    hw = h * w
    feats = features.reshape(bs, c, hw)
    lab = labels.astype(jnp.int32).reshape(bs, 1, hw)

    nj = hw // BLK
    nsteps = bs * nj

    out = pl.pallas_call(
        _proto_body(nsteps),
        grid=(bs, nj),
        in_specs=[
            pl.BlockSpec((1, c, BLK), lambda b, j: (b, 0, j)),
            pl.BlockSpec((1, 1, BLK), lambda b, j: (b, 0, j)),
        ],
        out_specs=pl.BlockSpec((c, OH_ROWS), lambda b, j: (0, 0)),
        out_shape=jax.ShapeDtypeStruct((c, OH_ROWS), jnp.float32),
    )(feats, lab)

    return out[:, :NUM_CLASSES - 1]


# speedup vs baseline: 1.1764x; 1.1764x over previous
import jax
import jax.numpy as jnp
from jax.experimental import pallas as pl

NUM_CLASSES = 20
OH_ROWS = 32

BLK = 4096


def _proto_body(nsteps):
    def body(f_ref, l_ref, o_ref):
        b = pl.program_id(0)
        j = pl.program_id(1)
        step = b * pl.num_programs(1) + j

        f = f_ref[0]
        lab = l_ref[0]

        sumsq = jnp.sum(f * f, axis=0, keepdims=True)
        invn = 1.0 / jnp.maximum(jnp.sqrt(sumsq), 1e-12)

        kiota = jax.lax.broadcasted_iota(jnp.int32, (OH_ROWS, BLK), 0)
        oh = jnp.where(kiota == lab, invn, 0.0)

        partial = jax.lax.dot_general(
            f, oh, (((1,), (1,)), ((), ())),
            preferred_element_type=jnp.float32)

        @pl.when(step == 0)
        def _():
            o_ref[...] = jnp.zeros_like(o_ref)

        o_ref[...] += partial

        @pl.when(step == nsteps - 1)
        def _():
            p = o_ref[...]
            pn = jnp.sqrt(jnp.sum(p * p, axis=0, keepdims=True))
            o_ref[...] = p / jnp.maximum(pn, 1e-12)

    return body


def kernel(features, labels):
    bs, c, h, w = features.shape
    hw = h * w
    feats = features.reshape(bs, c, hw)
    lab = labels.astype(jnp.int32).reshape(bs, 1, hw)

    nj = hw // BLK
    nsteps = bs * nj

    out = pl.pallas_call(
        _proto_body(nsteps),
        grid=(bs, nj),
        in_specs=[
            pl.BlockSpec((1, c, BLK), lambda b, j: (b, 0, j)),
            pl.BlockSpec((1, 1, BLK), lambda b, j: (b, 0, j)),
        ],
        out_specs=pl.BlockSpec((c, OH_ROWS), lambda b, j: (0, 0)),
        out_shape=jax.ShapeDtypeStruct((c, OH_ROWS), jnp.float32),
    )(feats, lab)

    return out[:, :NUM_CLASSES - 1]


# speedup vs baseline: 1.2968x; 1.1023x over previous
import jax
import jax.numpy as jnp
from jax.experimental import pallas as pl

NUM_CLASSES = 20
OH_ROWS = 32

BLK = 8192


def _proto_body(nsteps):
    def body(f_ref, l_ref, o_ref):
        b = pl.program_id(0)
        j = pl.program_id(1)
        step = b * pl.num_programs(1) + j

        f = f_ref[0]
        lab = l_ref[0]

        sumsq = jnp.sum(f * f, axis=0, keepdims=True)
        invn = 1.0 / jnp.maximum(jnp.sqrt(sumsq), 1e-12)

        kiota = jax.lax.broadcasted_iota(jnp.int32, (OH_ROWS, BLK), 0)
        oh = jnp.where(kiota == lab, invn, 0.0)

        partial = jax.lax.dot_general(
            f, oh, (((1,), (1,)), ((), ())),
            preferred_element_type=jnp.float32)

        @pl.when(step == 0)
        def _():
            o_ref[...] = jnp.zeros_like(o_ref)

        o_ref[...] += partial

        @pl.when(step == nsteps - 1)
        def _():
            p = o_ref[...]
            pn = jnp.sqrt(jnp.sum(p * p, axis=0, keepdims=True))
            o_ref[...] = p / jnp.maximum(pn, 1e-12)

    return body


def kernel(features, labels):
    bs, c, h, w = features.shape
    hw = h * w
    feats = features.reshape(bs, c, hw)
    lab = labels.astype(jnp.int32).reshape(bs, 1, hw)

    nj = hw // BLK
    nsteps = bs * nj

    out = pl.pallas_call(
        _proto_body(nsteps),
        grid=(bs, nj),
        in_specs=[
            pl.BlockSpec((1, c, BLK), lambda b, j: (b, 0, j)),
            pl.BlockSpec((1, 1, BLK), lambda b, j: (b, 0, j)),
        ],
        out_specs=pl.BlockSpec((c, OH_ROWS), lambda b, j: (0, 0)),
        out_shape=jax.ShapeDtypeStruct((c, OH_ROWS), jnp.float32),
    )(feats, lab)

    return out[:, :NUM_CLASSES - 1]


# speedup vs baseline: 1.3648x; 1.0524x over previous
import jax
import jax.numpy as jnp
from jax.experimental import pallas as pl

NUM_CLASSES = 20
OH_ROWS = 32

BLK = 16384


def _proto_body(nsteps):
    def body(f_ref, l_ref, o_ref):
        b = pl.program_id(0)
        j = pl.program_id(1)
        step = b * pl.num_programs(1) + j

        f = f_ref[0]
        lab = l_ref[0]

        sumsq = jnp.sum(f * f, axis=0, keepdims=True)
        invn = 1.0 / jnp.maximum(jnp.sqrt(sumsq), 1e-12)

        kiota = jax.lax.broadcasted_iota(jnp.int32, (OH_ROWS, BLK), 0)
        oh = jnp.where(kiota == lab, invn, 0.0)

        partial = jax.lax.dot_general(
            f, oh, (((1,), (1,)), ((), ())),
            preferred_element_type=jnp.float32)

        @pl.when(step == 0)
        def _():
            o_ref[...] = jnp.zeros_like(o_ref)

        o_ref[...] += partial

        @pl.when(step == nsteps - 1)
        def _():
            p = o_ref[...]
            pn = jnp.sqrt(jnp.sum(p * p, axis=0, keepdims=True))
            o_ref[...] = p / jnp.maximum(pn, 1e-12)

    return body


def kernel(features, labels):
    bs, c, h, w = features.shape
    hw = h * w
    feats = features.reshape(bs, c, hw)
    lab = labels.astype(jnp.int32).reshape(bs, 1, hw)

    nj = hw // BLK
    nsteps = bs * nj

    out = pl.pallas_call(
        _proto_body(nsteps),
        grid=(bs, nj),
        in_specs=[
            pl.BlockSpec((1, c, BLK), lambda b, j: (b, 0, j)),
            pl.BlockSpec((1, 1, BLK), lambda b, j: (b, 0, j)),
        ],
        out_specs=pl.BlockSpec((c, OH_ROWS), lambda b, j: (0, 0)),
        out_shape=jax.ShapeDtypeStruct((c, OH_ROWS), jnp.float32),
    )(feats, lab)

    return out[:, :NUM_CLASSES - 1]
